# baseline (speedup 1.0000x reference)
import dataclasses
import functools

import jax
import jax.numpy as jnp
from jax import lax
from jax.experimental import pallas as pl
from jax.experimental.pallas import tpu as pltpu
from jax.experimental.pallas import tpu_sc as plsc

_N_SUBCORES = 16
_CHUNK = 80
_EBATCH = 25


def _tc_transform(ns, w, a2):
    m, d = ns.shape
    units = w.shape[1]
    half = units // 2
    bm = 1000

    def body(ns_ref, w_ref, a_ref, h0_ref, h1_ref, s_ref):
        x = ns_ref[...]
        h = jnp.dot(x, w_ref[...], precision=lax.Precision.HIGHEST)
        h0_ref[...] = h[:, :half]
        h1_ref[...] = h[:, half:]
        s_ref[...] = lax.dot_general(
            h, a_ref[...], (((1,), (1,)), ((), ())),
            precision=lax.Precision.HIGHEST)

    return pl.pallas_call(
        body,
        grid=(m // bm,),
        in_specs=[
            pl.BlockSpec((bm, d), lambda i: (i, 0)),
            pl.BlockSpec((d, units), lambda i: (0, 0)),
            pl.BlockSpec((2, units), lambda i: (0, 0)),
        ],
        out_specs=[
            pl.BlockSpec((bm, half), lambda i: (i, 0)),
            pl.BlockSpec((bm, half), lambda i: (i, 0)),
            pl.BlockSpec((bm, 2), lambda i: (i, 0)),
        ],
        out_shape=[
            jax.ShapeDtypeStruct((m, half), jnp.float32),
            jax.ShapeDtypeStruct((m, half), jnp.float32),
            jax.ShapeDtypeStruct((m, 2), jnp.float32),
        ],
    )(ns, w, a2)


def _sc_gat(s_pair, h0, h1, edges5, n_nodes, n_pad):
    half = h0.shape[1]
    units = 2 * half
    n_batches = edges5.shape[1]
    n_chunks = n_batches * _EBATCH
    rows_per_tile = n_pad // _N_SUBCORES
    n_sub = rows_per_tile // _CHUNK
    mesh = plsc.VectorSubcoreMesh(core_axis_name="core", subcore_axis_name="subcore")
    cp = pltpu.CompilerParams()
    if "needs_layout_passes" in pltpu.CompilerParams.__dataclass_fields__:
        cp = dataclasses.replace(cp, needs_layout_passes=False)
    if "use_tc_tiling_on_sc" in pltpu.CompilerParams.__dataclass_fields__:
        cp = dataclasses.replace(cp, use_tc_tiling_on_sc=False)

    @functools.partial(
        pl.kernel,
        out_type=jax.ShapeDtypeStruct((n_nodes, units), jnp.float32),
        mesh=mesh,
        compiler_params=cp,
        scratch_types=[
            pltpu.VMEM((2 * n_nodes,), jnp.float32),
            pltpu.VMEM((n_chunks, _CHUNK), jnp.int32),
            pltpu.VMEM((n_chunks, _CHUNK), jnp.int32),
            pltpu.VMEM((2 * _EBATCH * _CHUNK,), jnp.int32),
            pltpu.VMEM((_CHUNK, half), jnp.float32),
            pltpu.VMEM((_CHUNK, half), jnp.float32),
            pltpu.VMEM((_CHUNK, half), jnp.float32),
            pltpu.VMEM((_CHUNK, half), jnp.float32),
            pltpu.VMEM((_CHUNK,), jnp.float32),
            pltpu.VMEM((_CHUNK,), jnp.float32),
            pltpu.VMEM((n_pad // _N_SUBCORES,), jnp.float32),
            pltpu.VMEM((n_pad // _N_SUBCORES,), jnp.float32),
            pltpu.VMEM_SHARED((n_pad,), jnp.float32),
            pltpu.VMEM_SHARED((n_pad, half), jnp.float32),
            pltpu.SemaphoreType.DMA,
            pltpu.SemaphoreType.DMA,
            pltpu.SemaphoreType.DMA,
            pltpu.SemaphoreType.DMA,
            pltpu.SemaphoreType.DMA,
            pltpu.SemaphoreType.DMA,
        ],
    )
    def k(s_hbm, h0_hbm, h1_hbm, e_hbm, out_hbm,
          s_v, dstv, srcv, ebuf, gbuf0, gbuf1, sbuf0, sbuf1, eb0, eb1,
          dn_v, rec_v, denom_sp, out_sp,
          gsem0, gsem1, ssem0, ssem1, dsem0, dsem1):
        core = lax.axis_index("core")
        t = lax.axis_index("subcore")
        base = t * rows_per_tile
        zeros16 = jnp.zeros((16,), jnp.float32)
        lane = lax.iota(jnp.int32, 16)

        pltpu.sync_copy(s_hbm, s_v)

        @pl.loop(0, n_batches)
        def _(bb):
            pltpu.sync_copy(e_hbm.at[t, bb], ebuf)

            @pl.loop(0, _EBATCH)
            def _(u):
                row = bb * _EBATCH + u
                ubase = jnp.full((16,), u * 2 * _CHUNK, jnp.int32)
                for g in range(_CHUNK // 16):
                    idx_d = ubase + (2 * (lane + g * 16))
                    d16 = plsc.load_gather(ebuf, [idx_d])
                    sx16 = plsc.load_gather(ebuf, [idx_d + 1])
                    dstv[row, pl.ds(g * 16, 16)] = d16
                    srcv[row, pl.ds(g * 16, 16)] = sx16

        @pl.loop(0, rows_per_tile // 16)
        def _(g):
            dn_v[pl.ds(g * 16, 16)] = zeros16

        pltpu.sync_copy(dn_v, denom_sp.at[pl.ds(base, rows_per_tile)])

        @pl.loop(0, _CHUNK)
        def _(j):
            for q in range(half // 16):
                sbuf0[j, pl.ds(q * 16, 16)] = zeros16

        @pl.loop(0, n_sub)
        def _(ksub):
            pltpu.sync_copy(sbuf0, out_sp.at[pl.ds(base + ksub * _CHUNK, _CHUNK)])

        plsc.subcore_barrier()

        bufs = ((gbuf0, gsem0, sbuf0, ssem0, eb0, dsem0),
                (gbuf1, gsem1, sbuf1, ssem1, eb1, dsem1))

        def main(h_ref):
            pltpu.async_copy(h_ref.at[srcv.at[0]], gbuf0, gsem0)
            pltpu.async_copy(h_ref.at[srcv.at[1]], gbuf1, gsem1)

            @pl.loop(0, n_chunks, step=2)
            def _(c):
                for b, (gbuf, gsem, sbuf, ssem, eb, dsem) in enumerate(bufs):
                    cc = c + b

                    @pl.when(c >= 2)
                    def _():
                        pltpu.make_async_copy(eb, denom_sp.at[dstv.at[cc]], dsem).wait()

                    for g in range(_CHUNK // 16):
                        d16 = dstv[cc, pl.ds(g * 16, 16)]
                        sx16 = srcv[cc, pl.ds(g * 16, 16)]
                        a = plsc.load_gather(s_v, [d16 * 2])
                        bsc = plsc.load_gather(s_v, [sx16 * 2 + 1])
                        x = a + bsc
                        x = jnp.maximum(x, x * 0.2)
                        x = jnp.minimum(jnp.maximum(x, -2.0), 2.0)
                        eb[pl.ds(g * 16, 16)] = jnp.exp(x)

                    pltpu.async_copy(eb, denom_sp.at[dstv.at[cc]], dsem, add=True)

                    pltpu.make_async_copy(h_ref.at[srcv.at[cc]], gbuf, gsem).wait()

                    @pl.when(c >= 2)
                    def _():
                        pltpu.make_async_copy(sbuf, out_sp.at[dstv.at[cc]], ssem).wait()

                    for g in range(_CHUNK // 16):
                        e16 = eb[pl.ds(g * 16, 16)]
                        for jj in range(16):
                            j = g * 16 + jj
                            ej = e16[jj]
                            for q in range(half // 16):
                                sbuf[j, pl.ds(q * 16, 16)] = gbuf[j, pl.ds(q * 16, 16)] * ej

                    @pl.when(cc + 2 < n_chunks)
                    def _():
                        pltpu.async_copy(h_ref.at[srcv.at[cc + 2]], gbuf, gsem)

                    pltpu.async_copy(sbuf, out_sp.at[dstv.at[cc]], ssem, add=True)

            for _b, (gbuf, gsem, sbuf, ssem, eb, dsem) in enumerate(bufs):
                pltpu.make_async_copy(sbuf, out_sp.at[dstv.at[0]], ssem).wait()
                pltpu.make_async_copy(eb, denom_sp.at[dstv.at[0]], dsem).wait()

        @pl.when(core == 0)
        def _():
            main(h0_hbm)

        @pl.when(core == 1)
        def _():
            main(h1_hbm)

        plsc.subcore_barrier()

        pltpu.sync_copy(denom_sp.at[pl.ds(base, rows_per_tile)], dn_v)

        @pl.loop(0, rows_per_tile // 16)
        def _(g):
            d16 = dn_v[pl.ds(g * 16, 16)]
            rec_v[pl.ds(g * 16, 16)] = 1.0 / jnp.maximum(d16, 1e-20)

        col = core * half

        @pl.loop(0, n_sub)
        def _(ksub):
            off = base + ksub * _CHUNK

            @pl.when(off < n_nodes)
            def _():
                pltpu.sync_copy(out_sp.at[pl.ds(off, _CHUNK)], gbuf0)

                for g in range(_CHUNK // 16):
                    r16 = rec_v[pl.ds(ksub * _CHUNK + g * 16, 16)]
                    for jj in range(16):
                        j = g * 16 + jj
                        rj = r16[jj]
                        for q in range(half // 16):
                            gbuf0[j, pl.ds(q * 16, 16)] = gbuf0[j, pl.ds(q * 16, 16)] * rj

                pltpu.sync_copy(gbuf0, out_hbm.at[pl.ds(off, _CHUNK), pl.ds(col, half)])

    return k(s_pair, h0, h1, edges5)


def kernel(node_states, edges, kernel, kernel_attention):
    n_nodes, _ = node_states.shape
    n_edges = edges.shape[0]
    n_pad = ((n_nodes + 16 * _CHUNK - 1) // (16 * _CHUNK)) * (16 * _CHUNK)
    per_tile = n_edges // _N_SUBCORES
    n_chunks = per_tile // _CHUNK
    n_batches = n_chunks // _EBATCH

    edges5 = edges.reshape(_N_SUBCORES, n_batches, 2 * _EBATCH * _CHUNK)
    a2 = kernel_attention.reshape(2, kernel.shape[1])

    h0, h1, s = _tc_transform(node_states, kernel, a2)
    return _sc_gat(s.reshape(-1), h0, h1, edges5, n_nodes, n_pad)
